# Initial kernel scaffold; baseline (speedup 1.0000x reference)
import jax
import jax.numpy as jnp
from jax.experimental import pallas as pl


def kernel(x, index, dim_size):
    raise NotImplementedError("write your pallas kernel here")


# speedup vs baseline: 3.6814x; 3.6814x over previous
import functools

import jax
import jax.numpy as jnp
from jax import lax
from jax.experimental import pallas as pl
from jax.experimental.pallas import tpu as pltpu
from jax.experimental.pallas import tpu_sc as plsc

_E = 320000
_D = 128
_N = 10000
_NC = 2
_NS = 16
_NW = _NC * _NS
_EPW = _E // _NW
_CHUNK = 80
_NCH = _EPW // _CHUNK
_ZR = 125
_RPT = _N // _NS
_OPT = 624


def _sc_body(x_hbm, idx_hbm, part_hbm, idx_v, x_v, z_v, acc):
    cid = lax.axis_index("c")
    sid = lax.axis_index("s")
    wid = cid * _NS + sid

    zero16 = jnp.zeros((16,), jnp.float32)

    def _zrow(r, c):
        def _zcol(j, cc):
            z_v[r, pl.ds(j * 16, 16)] = zero16
            return cc
        return lax.fori_loop(0, _D // 16, _zcol, c)

    lax.fori_loop(0, _ZR, _zrow, 0)

    for k in range(_RPT // _ZR):
        pltpu.sync_copy(z_v, acc.at[pl.ds(sid * _RPT + k * _ZR, _ZR), :])
    plsc.subcore_barrier()

    def _chunk(ch, c):
        base = wid * _EPW + ch * _CHUNK
        pltpu.sync_copy(idx_hbm.at[pl.ds(base, _CHUNK)], idx_v)
        pltpu.sync_copy(x_hbm.at[pl.ds(base, _CHUNK), :], x_v)
        pltpu.sync_copy(x_v, acc.at[idx_v], add=True)
        return c

    lax.fori_loop(0, _NCH, _chunk, 0)
    plsc.subcore_barrier()

    pltpu.sync_copy(
        acc.at[pl.ds(sid * _OPT, _OPT), :],
        part_hbm.at[cid, pl.ds(sid * _OPT, _OPT), :],
    )

    @pl.when(sid == _NS - 1)
    def _tail():
        pltpu.sync_copy(
            acc.at[pl.ds(_NS * _OPT, _N - _NS * _OPT), :],
            part_hbm.at[cid, pl.ds(_NS * _OPT, _N - _NS * _OPT), :],
        )


_sc_scatter = functools.partial(
    pl.kernel,
    out_type=jax.ShapeDtypeStruct((_NC, _N, _D), jnp.float32),
    mesh=plsc.VectorSubcoreMesh(core_axis_name="c", subcore_axis_name="s"),
    scratch_types=[
        pltpu.VMEM((_CHUNK,), jnp.int32),
        pltpu.VMEM((_CHUNK, _D), jnp.float32),
        pltpu.VMEM((_ZR, _D), jnp.float32),
        pltpu.VMEM_SHARED((_N, _D), jnp.float32),
    ],
)(_sc_body)


def _add_body(a_ref, b_ref, o_ref):
    o_ref[...] = a_ref[...] + b_ref[...]


def _sum_partials(p0, p1):
    return pl.pallas_call(
        _add_body,
        grid=(10,),
        in_specs=[
            pl.BlockSpec((_N // 10, _D), lambda i: (i, 0)),
            pl.BlockSpec((_N // 10, _D), lambda i: (i, 0)),
        ],
        out_specs=pl.BlockSpec((_N // 10, _D), lambda i: (i, 0)),
        out_shape=jax.ShapeDtypeStruct((_N, _D), jnp.float32),
    )(p0, p1)


def kernel(x, index, dim_size):
    del dim_size
    idx32 = index.astype(jnp.int32)
    parts = _sc_scatter(x, idx32)
    return _sum_partials(parts[0], parts[1])


# speedup vs baseline: 7.9550x; 2.1609x over previous
import functools

import jax
import jax.numpy as jnp
from jax import lax
from jax.experimental import pallas as pl
from jax.experimental.pallas import tpu as pltpu
from jax.experimental.pallas import tpu_sc as plsc

_E = 320000
_D = 128
_N = 10000
_NC = 2
_NS = 16
_NW = _NC * _NS
_EPW = _E // _NW
_CHUNK = 40
_NCH = _EPW // _CHUNK
_NB = 5
_RPT = _N // _NS
_OPT = 624


def _sc_body(x_hbm, idx_hbm, part_hbm, idx_v, x_v, acc, sem_in, sem_sc):
    cid = lax.axis_index("c")
    sid = lax.axis_index("s")
    wid = cid * _NS + sid
    e0 = wid * _EPW

    def _start_load(ch, b):
        base = e0 + ch * _CHUNK
        pltpu.async_copy(idx_hbm.at[pl.ds(base, _CHUNK)], idx_v.at[b], sem_in.at[b])
        pltpu.async_copy(x_hbm.at[pl.ds(base, _CHUNK), :], x_v.at[b], sem_in.at[b])

    def _wait_load(b):
        pltpu.make_async_copy(idx_hbm.at[pl.ds(0, _CHUNK)], idx_v.at[b], sem_in.at[b]).wait()
        pltpu.make_async_copy(x_hbm.at[pl.ds(0, _CHUNK), :], x_v.at[b], sem_in.at[b]).wait()

    def _scatter_desc(b):
        return pltpu.make_async_copy(x_v.at[b], acc.at[idx_v.at[b]], sem_sc.at[b])

    for b in range(_NB - 1):
        _start_load(b, b)

    zero16 = jnp.zeros((16,), jnp.float32)

    def _zrow(r, c):
        def _zcol(j, cc):
            x_v[_NB - 1, r, pl.ds(j * 16, 16)] = zero16
            return cc
        return lax.fori_loop(0, _D // 16, _zcol, c)

    lax.fori_loop(0, _CHUNK, _zrow, 0)

    for k in range(_RPT // _CHUNK):
        pltpu.sync_copy(
            x_v.at[_NB - 1], acc.at[pl.ds(sid * _RPT + k * _CHUNK, _CHUNK), :]
        )
    _ZT = _RPT % _CHUNK
    pltpu.sync_copy(
        x_v.at[_NB - 1, pl.ds(0, _ZT), :],
        acc.at[pl.ds(sid * _RPT + _RPT - _ZT, _ZT), :],
    )
    plsc.subcore_barrier()

    def _group(j, carry):
        for b in range(_NB):
            c = j * _NB + b
            bp = (b + _NB - 1) % _NB
            pf = c + _NB - 1

            @pl.when((c >= 1) & (pf < _NCH))
            def _drain_prev():
                _scatter_desc(bp).wait()

            @pl.when(pf < _NCH)
            def _prefetch():
                _start_load(pf, bp)

            _wait_load(b)
            pltpu.async_copy(x_v.at[b], acc.at[idx_v.at[b]], sem_sc.at[b], add=True)
        return carry

    lax.fori_loop(0, _NCH // _NB, _group, 0)

    for b in range(_NB):
        _scatter_desc(b).wait()
    plsc.subcore_barrier()

    pltpu.sync_copy(
        acc.at[pl.ds(sid * _OPT, _OPT), :],
        part_hbm.at[cid, pl.ds(sid * _OPT, _OPT), :],
    )

    @pl.when(sid == _NS - 1)
    def _tail():
        pltpu.sync_copy(
            acc.at[pl.ds(_NS * _OPT, _N - _NS * _OPT), :],
            part_hbm.at[cid, pl.ds(_NS * _OPT, _N - _NS * _OPT), :],
        )


_sc_scatter = functools.partial(
    pl.kernel,
    out_type=jax.ShapeDtypeStruct((_NC, _N, _D), jnp.float32),
    mesh=plsc.VectorSubcoreMesh(core_axis_name="c", subcore_axis_name="s"),
    scratch_types=[
        pltpu.VMEM((_NB, _CHUNK), jnp.int32),
        pltpu.VMEM((_NB, _CHUNK, _D), jnp.float32),
        pltpu.VMEM_SHARED((_N, _D), jnp.float32),
        pltpu.SemaphoreType.DMA((_NB,)),
        pltpu.SemaphoreType.DMA((_NB,)),
    ],
)(_sc_body)


def _add_body(a_ref, b_ref, o_ref):
    o_ref[...] = a_ref[...] + b_ref[...]


def _sum_partials(p0, p1):
    return pl.pallas_call(
        _add_body,
        grid=(10,),
        in_specs=[
            pl.BlockSpec((_N // 10, _D), lambda i: (i, 0)),
            pl.BlockSpec((_N // 10, _D), lambda i: (i, 0)),
        ],
        out_specs=pl.BlockSpec((_N // 10, _D), lambda i: (i, 0)),
        out_shape=jax.ShapeDtypeStruct((_N, _D), jnp.float32),
    )(p0, p1)


def kernel(x, index, dim_size):
    del dim_size
    idx32 = index.astype(jnp.int32)
    parts = _sc_scatter(x, idx32)
    return _sum_partials(parts[0], parts[1])


# speedup vs baseline: 7.9745x; 1.0024x over previous
import functools

import jax
import jax.numpy as jnp
from jax import lax
from jax.experimental import pallas as pl
from jax.experimental.pallas import tpu as pltpu
from jax.experimental.pallas import tpu_sc as plsc

_E = 320000
_D = 128
_N = 10000
_NC = 2
_NS = 16
_NW = _NC * _NS
_EPW = _E // _NW
_CHUNK = 80
_NCH = _EPW // _CHUNK
_NB = 4
_RPT = _N // _NS
_OPT = 624


def _sc_body(x_hbm, idx_hbm, part_hbm, idx_v, x_v, acc, sem_in, sem_sc):
    cid = lax.axis_index("c")
    sid = lax.axis_index("s")
    wid = cid * _NS + sid
    e0 = wid * _EPW

    def _start_load(ch, b):
        base = e0 + ch * _CHUNK
        pltpu.async_copy(idx_hbm.at[pl.ds(base, _CHUNK)], idx_v.at[b], sem_in.at[b])
        pltpu.async_copy(x_hbm.at[pl.ds(base, _CHUNK), :], x_v.at[b], sem_in.at[b])

    def _wait_load(b):
        pltpu.make_async_copy(idx_hbm.at[pl.ds(0, _CHUNK)], idx_v.at[b], sem_in.at[b]).wait()
        pltpu.make_async_copy(x_hbm.at[pl.ds(0, _CHUNK), :], x_v.at[b], sem_in.at[b]).wait()

    def _scatter_desc(b):
        return pltpu.make_async_copy(x_v.at[b], acc.at[idx_v.at[b]], sem_sc.at[b])

    for b in range(_NB - 1):
        _start_load(b, b)

    zero16 = jnp.zeros((16,), jnp.float32)

    def _zrow(r, c):
        def _zcol(j, cc):
            x_v[_NB - 1, r, pl.ds(j * 16, 16)] = zero16
            return cc
        return lax.fori_loop(0, _D // 16, _zcol, c)

    lax.fori_loop(0, _CHUNK, _zrow, 0)

    for k in range(_RPT // _CHUNK):
        pltpu.sync_copy(
            x_v.at[_NB - 1], acc.at[pl.ds(sid * _RPT + k * _CHUNK, _CHUNK), :]
        )
    _ZT = _RPT % _CHUNK
    pltpu.sync_copy(
        x_v.at[_NB - 1, pl.ds(0, _ZT), :],
        acc.at[pl.ds(sid * _RPT + _RPT - _ZT, _ZT), :],
    )
    plsc.subcore_barrier()

    def _group(j, carry):
        for b in range(_NB):
            c = j * _NB + b
            bp = (b + _NB - 1) % _NB
            pf = c + _NB - 1

            @pl.when((c >= 1) & (pf < _NCH))
            def _drain_prev():
                _scatter_desc(bp).wait()

            @pl.when(pf < _NCH)
            def _prefetch():
                _start_load(pf, bp)

            @pl.when(c < _NCH)
            def _consume():
                _wait_load(b)
                pltpu.async_copy(
                    x_v.at[b], acc.at[idx_v.at[b]], sem_sc.at[b], add=True
                )
        return carry

    lax.fori_loop(0, (_NCH + _NB - 1) // _NB, _group, 0)

    for b in range(_NB):
        _scatter_desc(b).wait()
    plsc.subcore_barrier()

    pltpu.sync_copy(
        acc.at[pl.ds(sid * _OPT, _OPT), :],
        part_hbm.at[cid, pl.ds(sid * _OPT, _OPT), :],
    )

    @pl.when(sid == _NS - 1)
    def _tail():
        pltpu.sync_copy(
            acc.at[pl.ds(_NS * _OPT, _N - _NS * _OPT), :],
            part_hbm.at[cid, pl.ds(_NS * _OPT, _N - _NS * _OPT), :],
        )


_sc_scatter = functools.partial(
    pl.kernel,
    out_type=jax.ShapeDtypeStruct((_NC, _N, _D), jnp.float32),
    mesh=plsc.VectorSubcoreMesh(core_axis_name="c", subcore_axis_name="s"),
    scratch_types=[
        pltpu.VMEM((_NB, _CHUNK), jnp.int32),
        pltpu.VMEM((_NB, _CHUNK, _D), jnp.float32),
        pltpu.VMEM_SHARED((_N, _D), jnp.float32),
        pltpu.SemaphoreType.DMA((_NB,)),
        pltpu.SemaphoreType.DMA((_NB,)),
    ],
)(_sc_body)


def _add_body(a_ref, b_ref, o_ref):
    o_ref[...] = a_ref[...] + b_ref[...]


def _sum_partials(p0, p1):
    return pl.pallas_call(
        _add_body,
        grid=(10,),
        in_specs=[
            pl.BlockSpec((_N // 10, _D), lambda i: (i, 0)),
            pl.BlockSpec((_N // 10, _D), lambda i: (i, 0)),
        ],
        out_specs=pl.BlockSpec((_N // 10, _D), lambda i: (i, 0)),
        out_shape=jax.ShapeDtypeStruct((_N, _D), jnp.float32),
    )(p0, p1)


def kernel(x, index, dim_size):
    del dim_size
    idx32 = index.astype(jnp.int32)
    parts = _sc_scatter(x, idx32)
    return _sum_partials(parts[0], parts[1])


# speedup vs baseline: 8.4128x; 1.0550x over previous
import functools

import jax
import jax.numpy as jnp
from jax import lax
from jax.experimental import pallas as pl
from jax.experimental.pallas import tpu as pltpu
from jax.experimental.pallas import tpu_sc as plsc

_E = 320000
_D = 128
_N = 10000
_HN = _N // 2
_NC = 2
_NS = 16
_CH = 64
_NCHT = _E // _CH
_NB = 5
_AN = _HN + 8
_ZPT = _AN // _NS
_OPT = 312
_SPT = _E // _NS


def _sc_body(x_hbm, idx_hbm, out_hbm, idx_v, x_v, scan_v, cx_v, cxr_v, acc,
             cnts_sh, sem_in, sem_sc):
    cid = lax.axis_index("c")
    sid = lax.axis_index("s")

    pltpu.sync_copy(idx_hbm.at[pl.ds(sid * _SPT, _SPT)], scan_v)

    def _count(i, cvec):
        v = scan_v[pl.ds(i * 16, 16)]
        return cvec + jnp.where(v < _HN, 1, 0).astype(jnp.int32)

    cvec = lax.fori_loop(0, _SPT // 16, _count, jnp.zeros((16,), jnp.int32))
    cx_v[...] = cvec
    pltpu.sync_copy(cx_v, cnts_sh.at[pl.ds(sid * 16, 16)])

    zero16 = jnp.zeros((16,), jnp.float32)

    def _zrow(r, c):
        def _zcol(j, cc):
            x_v[_NB - 1, r, pl.ds(j * 16, 16)] = zero16
            return cc
        return lax.fori_loop(0, _D // 16, _zcol, c)

    lax.fori_loop(0, _CH, _zrow, 0)

    for k in range(_ZPT // _CH):
        pltpu.sync_copy(x_v.at[_NB - 1], acc.at[pl.ds(sid * _ZPT + k * _CH, _CH), :])
    _zt = _ZPT % _CH
    pltpu.sync_copy(
        x_v.at[_NB - 1, pl.ds(0, _zt), :],
        acc.at[pl.ds(sid * _ZPT + _ZPT - _zt, _zt), :],
    )
    plsc.subcore_barrier()

    pltpu.sync_copy(cnts_sh, cxr_v)
    svec = cxr_v[pl.ds(0, 16)]
    for r in range(1, _NS):
        svec = svec + cxr_v[pl.ds(r * 16, 16)]
    s = jnp.int32(0)
    for q in range(16):
        s = s + svec[q]
    s = jnp.minimum(jnp.maximum(s, 0), _E)

    c0 = (s + _CH - 1) // _CH
    c1 = s // _CH
    m = jnp.where(cid == 0, c0, _NCHT - c1)
    base_c = jnp.where(cid == 0, 0, c1)
    lo = base_c + (m * sid) // _NS
    hi = base_c + (m * (sid + 1)) // _NS
    kstop = hi - lo

    def _start_load(c, b):
        pos = jnp.maximum(lo + jnp.minimum(c, kstop - 1), 0)
        pltpu.async_copy(
            idx_hbm.at[pl.ds(pos * _CH, _CH)], idx_v.at[b], sem_in.at[b])
        pltpu.async_copy(
            x_hbm.at[pl.ds(pos * _CH, _CH), :], x_v.at[b], sem_in.at[b])

    def _wait_load(b):
        pltpu.make_async_copy(
            idx_hbm.at[pl.ds(0, _CH)], idx_v.at[b], sem_in.at[b]).wait()
        pltpu.make_async_copy(
            x_hbm.at[pl.ds(0, _CH), :], x_v.at[b], sem_in.at[b]).wait()

    def _scatter_desc(b):
        return pltpu.make_async_copy(x_v.at[b], acc.at[idx_v.at[b]], sem_sc.at[b])

    base_row = cid * _HN

    def _consume(c, b):
        _wait_load(b)
        voff = jnp.full(
            (16,), base_row - jnp.where(c < kstop, 0, 2 * _HN), jnp.int32)
        for q in range(_CH // 16):
            v = idx_v[b, pl.ds(q * 16, 16)]
            loc = v - voff
            ok = (loc >= 0) & (loc < _HN)
            idx_v[b, pl.ds(q * 16, 16)] = jnp.where(ok, loc, _HN)
        pltpu.async_copy(x_v.at[b], acc.at[idx_v.at[b]], sem_sc.at[b], add=True)

    for b in range(_NB - 1):
        _start_load(b, b)

    for b in range(_NB):
        bp = (b + _NB - 1) % _NB
        if b >= 1:
            _scatter_desc(bp).wait()
        _start_load(b + _NB - 1, bp)
        _consume(b, b)

    def _group(j, carry):
        for b in range(_NB):
            c = j * _NB + b
            bp = (b + _NB - 1) % _NB
            _scatter_desc(bp).wait()
            _start_load(c + _NB - 1, bp)
            _consume(c, b)
        return carry

    n_groups = jnp.maximum((kstop + _NB - 1) // _NB, 1)
    lax.fori_loop(1, n_groups, _group, 0)

    _scatter_desc(_NB - 1).wait()
    for b in range(_NB - 1):
        _wait_load(b)
    plsc.subcore_barrier()

    pltpu.sync_copy(
        acc.at[pl.ds(sid * _OPT, _OPT), :],
        out_hbm.at[pl.ds(base_row + sid * _OPT, _OPT), :],
    )

    @pl.when(sid == _NS - 1)
    def _tail():
        pltpu.sync_copy(
            acc.at[pl.ds(_NS * _OPT, _HN - _NS * _OPT), :],
            out_hbm.at[pl.ds(base_row + _NS * _OPT, _HN - _NS * _OPT), :],
        )


_sc_scatter = functools.partial(
    pl.kernel,
    out_type=jax.ShapeDtypeStruct((_N, _D), jnp.float32),
    mesh=plsc.VectorSubcoreMesh(core_axis_name="c", subcore_axis_name="s"),
    scratch_types=[
        pltpu.VMEM((_NB, _CH), jnp.int32),
        pltpu.VMEM((_NB, _CH, _D), jnp.float32),
        pltpu.VMEM((_SPT,), jnp.int32),
        pltpu.VMEM((16,), jnp.int32),
        pltpu.VMEM((_NS * 16,), jnp.int32),
        pltpu.VMEM_SHARED((_AN, _D), jnp.float32),
        pltpu.VMEM_SHARED((_NS * 16,), jnp.int32),
        pltpu.SemaphoreType.DMA((_NB,)),
        pltpu.SemaphoreType.DMA((_NB,)),
    ],
)(_sc_body)


def kernel(x, index, dim_size):
    del dim_size
    return _sc_scatter(x, index.astype(jnp.int32))


# speedup vs baseline: 8.7111x; 1.0355x over previous
import functools

import jax
import jax.numpy as jnp
from jax import lax
from jax.experimental import pallas as pl
from jax.experimental.pallas import tpu as pltpu
from jax.experimental.pallas import tpu_sc as plsc

_E = 320000
_D = 128
_N = 10000
_HN = _N // 2
_NC = 2
_NS = 16
_CH = 64
_NCHT = _E // _CH
_NB = 5
_AN = _HN + 8
_ZPT = _AN // _NS
_OPT = 312
_SPT = _E // _NS


def _sc_body(x_hbm, idx_hbm, out_hbm, idx_v, x_v, scan_v, cx_v, cxr_v, acc,
             cnts_sh, sem_in, sem_sc):
    cid = lax.axis_index("c")
    sid = lax.axis_index("s")

    pltpu.sync_copy(idx_hbm.at[pl.ds(sid * _SPT, _SPT)], scan_v.at[pl.ds(0, _SPT)])

    def _bstep(_, lohi):
        blo, bhi = lohi
        mid = (blo + bhi) // 2
        v = scan_v[pl.ds(mid, 16)]
        open_ = blo < bhi
        pred = jnp.logical_and(open_, v[0] < _HN)
        return (jnp.where(pred, mid + 1, blo),
                jnp.where(jnp.logical_and(open_, jnp.logical_not(pred)), mid, bhi))

    cnt, _ = lax.fori_loop(
        0, 15, _bstep, (jnp.int32(0), jnp.int32(_SPT)))
    cx_v[...] = jnp.full((16,), cnt, jnp.int32)
    pltpu.sync_copy(cx_v, cnts_sh.at[pl.ds(sid * 16, 16)])

    zero16 = jnp.zeros((16,), jnp.float32)

    def _zrow(r, c):
        def _zcol(j, cc):
            x_v[_NB - 1, r, pl.ds(j * 16, 16)] = zero16
            return cc
        return lax.fori_loop(0, _D // 16, _zcol, c)

    lax.fori_loop(0, _CH, _zrow, 0)

    for k in range(_ZPT // _CH):
        pltpu.sync_copy(x_v.at[_NB - 1], acc.at[pl.ds(sid * _ZPT + k * _CH, _CH), :])
    _zt = _ZPT % _CH
    pltpu.sync_copy(
        x_v.at[_NB - 1, pl.ds(0, _zt), :],
        acc.at[pl.ds(sid * _ZPT + _ZPT - _zt, _zt), :],
    )
    plsc.subcore_barrier()

    pltpu.sync_copy(cnts_sh, cxr_v)
    svec = cxr_v[pl.ds(0, 16)]
    for r in range(1, _NS):
        svec = svec + cxr_v[pl.ds(r * 16, 16)]
    s = jnp.minimum(jnp.maximum(svec[0], 0), _E)

    c0 = (s + _CH - 1) // _CH
    c1 = s // _CH
    m = jnp.where(cid == 0, c0, _NCHT - c1)
    base_c = jnp.where(cid == 0, 0, c1)
    lo = base_c + (m * sid) // _NS
    hi = base_c + (m * (sid + 1)) // _NS
    kstop = hi - lo

    def _start_load(c, b):
        pos = jnp.maximum(lo + jnp.minimum(c, kstop - 1), 0)
        pltpu.async_copy(
            idx_hbm.at[pl.ds(pos * _CH, _CH)], idx_v.at[b], sem_in.at[b])
        pltpu.async_copy(
            x_hbm.at[pl.ds(pos * _CH, _CH), :], x_v.at[b], sem_in.at[b])

    def _wait_load(b):
        pltpu.make_async_copy(
            idx_hbm.at[pl.ds(0, _CH)], idx_v.at[b], sem_in.at[b]).wait()
        pltpu.make_async_copy(
            x_hbm.at[pl.ds(0, _CH), :], x_v.at[b], sem_in.at[b]).wait()

    def _scatter_desc(b):
        return pltpu.make_async_copy(x_v.at[b], acc.at[idx_v.at[b]], sem_sc.at[b])

    base_row = cid * _HN

    def _consume(c, b):
        _wait_load(b)
        voff = jnp.full(
            (16,), base_row - jnp.where(c < kstop, 0, 2 * _HN), jnp.int32)
        for q in range(_CH // 16):
            v = idx_v[b, pl.ds(q * 16, 16)]
            loc = jnp.minimum(
                (v - voff).astype(jnp.uint32), jnp.uint32(_HN))
            idx_v[b, pl.ds(q * 16, 16)] = loc.astype(jnp.int32)
        pltpu.async_copy(x_v.at[b], acc.at[idx_v.at[b]], sem_sc.at[b], add=True)

    for b in range(_NB - 1):
        _start_load(b, b)

    for b in range(_NB):
        bp = (b + _NB - 1) % _NB
        if b >= 1:
            _scatter_desc(bp).wait()
        _start_load(b + _NB - 1, bp)
        _consume(b, b)

    def _group(j, carry):
        for b in range(_NB):
            c = j * _NB + b
            bp = (b + _NB - 1) % _NB
            _scatter_desc(bp).wait()
            _start_load(c + _NB - 1, bp)
            _consume(c, b)
        return carry

    n_groups = jnp.maximum((kstop + _NB - 1) // _NB, 1)
    lax.fori_loop(1, n_groups, _group, 0)

    _scatter_desc(_NB - 1).wait()
    for b in range(_NB - 1):
        _wait_load(b)
    plsc.subcore_barrier()

    pltpu.sync_copy(
        acc.at[pl.ds(sid * _OPT, _OPT), :],
        out_hbm.at[pl.ds(base_row + sid * _OPT, _OPT), :],
    )

    @pl.when(sid == _NS - 1)
    def _tail():
        pltpu.sync_copy(
            acc.at[pl.ds(_NS * _OPT, _HN - _NS * _OPT), :],
            out_hbm.at[pl.ds(base_row + _NS * _OPT, _HN - _NS * _OPT), :],
        )


_sc_scatter = functools.partial(
    pl.kernel,
    out_type=jax.ShapeDtypeStruct((_N, _D), jnp.float32),
    mesh=plsc.VectorSubcoreMesh(core_axis_name="c", subcore_axis_name="s"),
    scratch_types=[
        pltpu.VMEM((_NB, _CH), jnp.int32),
        pltpu.VMEM((_NB, _CH, _D), jnp.float32),
        pltpu.VMEM((_SPT + 16,), jnp.int32),
        pltpu.VMEM((16,), jnp.int32),
        pltpu.VMEM((_NS * 16,), jnp.int32),
        pltpu.VMEM_SHARED((_AN, _D), jnp.float32),
        pltpu.VMEM_SHARED((_NS * 16,), jnp.int32),
        pltpu.SemaphoreType.DMA((_NB,)),
        pltpu.SemaphoreType.DMA((_NB,)),
    ],
)(_sc_body)


def kernel(x, index, dim_size):
    del dim_size
    return _sc_scatter(x, index.astype(jnp.int32))
